# Initial kernel scaffold; baseline (speedup 1.0000x reference)
import jax
import jax.numpy as jnp
from jax.experimental import pallas as pl


def kernel(src, index, dim_size):
    raise NotImplementedError("write your pallas kernel here")


# speedup vs baseline: 7.1932x; 7.1932x over previous
import functools

import jax
import jax.numpy as jnp
from jax import lax
from jax.experimental import pallas as pl
from jax.experimental.pallas import tpu as pltpu
from jax.experimental.pallas import tpu_sc as plsc

NSEG = 10000
D = 128
ROWS = 320000
NC = 2
NS = 16
NW = NC * NS
RPW = ROWS // NW
CH = 80
NCHUNK = RPW // CH
NSEG_PAD = 10240
SEG_PER_TILE = NSEG_PAD // NS
ZROWS = 16


def _sc_partial_segsum(src, idx3d):
    mesh = plsc.VectorSubcoreMesh(core_axis_name="c", subcore_axis_name="s")

    @functools.partial(
        pl.kernel,
        out_type=jax.ShapeDtypeStruct((NC, NSEG_PAD, D), jnp.float32),
        mesh=mesh,
        scratch_types=[
            pltpu.VMEM((CH, D), jnp.float32),
            pltpu.VMEM((CH, D), jnp.float32),
            pltpu.VMEM((NCHUNK, CH), jnp.int32),
            pltpu.VMEM_SHARED((NSEG_PAD, D), jnp.float32),
            pltpu.SemaphoreType.DMA,
            pltpu.SemaphoreType.DMA,
        ],
    )
    def k(src_hbm, idx_hbm, out_hbm, rows0, rows1, idx_v, acc_sh, sem0, sem1):
        c = lax.axis_index("c")
        s = lax.axis_index("s")
        wid = c * NS + s
        row0 = wid * RPW

        zeros16 = jnp.zeros((16,), jnp.float32)
        for i in range(ZROWS):
            for j in range(D // 16):
                rows0[i, pl.ds(j * 16, 16)] = zeros16
        for i in range(SEG_PER_TILE // ZROWS):
            pltpu.sync_copy(
                rows0.at[pl.ds(0, ZROWS)],
                acc_sh.at[pl.ds(s * SEG_PER_TILE + i * ZROWS, ZROWS)],
            )

        pltpu.sync_copy(idx_hbm.at[wid], idx_v)
        plsc.subcore_barrier()

        def load(g, buf, sem):
            base = pl.multiple_of(row0 + g * CH, CH)
            return pltpu.make_async_copy(src_hbm.at[pl.ds(base, CH)], buf, sem)

        def scatter(g, buf):
            pltpu.sync_copy(buf, acc_sh.at[idx_v.at[g]], add=True)

        load(0, rows0, sem0).start()

        def body(i, _):
            g0 = 2 * i
            load(g0 + 1, rows1, sem1).start()
            load(g0, rows0, sem0).wait()
            scatter(g0, rows0)
            load(g0 + 2, rows0, sem0).start()
            load(g0 + 1, rows1, sem1).wait()
            scatter(g0 + 1, rows1)
            return 0

        lax.fori_loop(0, (NCHUNK - 1) // 2, body, 0)
        load(NCHUNK - 1, rows0, sem0).wait()
        scatter(NCHUNK - 1, rows0)
        plsc.subcore_barrier()

        pltpu.sync_copy(
            acc_sh.at[pl.ds(s * SEG_PER_TILE, SEG_PER_TILE)],
            out_hbm.at[c, pl.ds(s * SEG_PER_TILE, SEG_PER_TILE)],
        )

    return k(src, idx3d)


def _tc_add_partials(partials):
    def body(p_ref, o_ref):
        o_ref[...] = p_ref[0] + p_ref[1]

    blk = NSEG // 10
    return pl.pallas_call(
        body,
        out_shape=jax.ShapeDtypeStruct((NSEG, D), jnp.float32),
        grid=(NSEG // blk,),
        in_specs=[pl.BlockSpec((NC, blk, D), lambda i: (0, i, 0))],
        out_specs=pl.BlockSpec((blk, D), lambda i: (i, 0)),
    )(partials)


def kernel(src, index, dim_size):
    idx = jnp.minimum(index, dim_size - 1).astype(jnp.int32)
    idx3d = idx.reshape(NW, NCHUNK, CH)
    partials = _sc_partial_segsum(src, idx3d)
    return _tc_add_partials(partials)
